# baseline (speedup 1.0000x reference)
import functools
import jax
import jax.numpy as jnp
from jax import lax
from jax.experimental import pallas as pl
from jax.experimental.pallas import tpu as pltpu
from jax.experimental.pallas import tpu_sc as plsc

N = 10000
DEG = 32
H = 8
DH = 16
HDH = H * DH
FTW = DEG * HDH
A2W = DEG * H

NWORK = 32
C = 2
SPAN = 320
NC = SPAN // C


def _sc_body(a1_hbm, a2_hbm, ft_hbm, out_hbm,
             ftb, a2b, a1b, outb, rsc, insem, osem):
    cid = lax.axis_index("c")
    sid = lax.axis_index("s")
    wid = sid * 2 + cid
    start = (wid * (N - SPAN)) // (NWORK - 1)

    idx8 = lax.rem(lax.iota(jnp.int32, 16) + 8, 16)

    def issue_in(chunk, b):
        base = start + chunk * C
        pltpu.make_async_copy(
            ft_hbm.at[pl.ds(base, C), :], ftb.at[b], insem.at[b]).start()
        pltpu.make_async_copy(
            a2_hbm.at[pl.ds(base, C), :], a2b.at[b], insem.at[b]).start()
        pltpu.make_async_copy(
            a1_hbm.at[pl.ds(base, C), :], a1b.at[b], insem.at[b]).start()

    def wait_in(b):
        pltpu.make_async_copy(
            ft_hbm.at[pl.ds(0, C), :], ftb.at[b], insem.at[b]).wait()
        pltpu.make_async_copy(
            a2_hbm.at[pl.ds(0, C), :], a2b.at[b], insem.at[b]).wait()
        pltpu.make_async_copy(
            a1_hbm.at[pl.ds(0, C), :], a1b.at[b], insem.at[b]).wait()

    def compute_node(b, i):
        a1v = a1b[b, i, :]
        u = []
        for k in range(16):
            v = a2b[b, i, pl.ds(16 * k, 16)] + a1v
            u.append(jnp.maximum(v, 0.01 * v))
        m = u[0]
        for k in range(1, 16):
            m = jnp.maximum(m, u[k])
        rsc[:] = m
        m = jnp.maximum(m, plsc.load_gather(rsc, [idx8]))
        e = [jnp.exp(u[k] - m) for k in range(16)]
        s = e[0]
        for k in range(1, 16):
            s = s + e[k]
        rsc[:] = s
        s = s + plsc.load_gather(rsc, [idx8])
        r = 1.0 / s
        w = [e[k] * r for k in range(16)]
        for h in range(H):
            acc = w[0][h] * ftb[b, i, pl.ds(h * DH, 16)]
            for d in range(1, DEG):
                ws = w[d // 2][h + 8 * (d % 2)]
                acc = acc + ws * ftb[b, i, pl.ds(d * HDH + h * DH, 16)]
            outb[b, i, pl.ds(h * DH, 16)] = acc

    issue_in(0, 0)

    def outer(it0, carry):
        for b in range(2):
            chunk = it0 * 2 + b

            @pl.when(chunk + 1 < NC)
            def _():
                issue_in(chunk + 1, 1 - b)

            wait_in(b)

            @pl.when(chunk >= 2)
            def _():
                pltpu.make_async_copy(
                    outb.at[b], out_hbm.at[pl.ds(0, C), :], osem.at[b]).wait()

            for i in range(C):
                compute_node(b, i)

            base = start + chunk * C
            pltpu.make_async_copy(
                outb.at[b], out_hbm.at[pl.ds(base, C), :], osem.at[b]).start()
        return carry

    lax.fori_loop(0, NC // 2, outer, 0)
    for b in range(2):
        pltpu.make_async_copy(
            outb.at[b], out_hbm.at[pl.ds(0, C), :], osem.at[b]).wait()


@functools.partial(jax.jit, static_argnums=())
def _sc_call(a1p, a2p, ftr):
    mesh = plsc.VectorSubcoreMesh(core_axis_name="c", subcore_axis_name="s")
    k = pl.kernel(
        _sc_body,
        out_type=jax.ShapeDtypeStruct((N, HDH), jnp.float32),
        mesh=mesh,
        compiler_params=pltpu.CompilerParams(use_tc_tiling_on_sc=False, needs_layout_passes=False),
        scratch_types=[
            pltpu.VMEM((2, C, FTW), jnp.float32),
            pltpu.VMEM((2, C, A2W), jnp.float32),
            pltpu.VMEM((2, C, 16), jnp.float32),
            pltpu.VMEM((2, C, HDH), jnp.float32),
            pltpu.VMEM((16,), jnp.float32),
            pltpu.SemaphoreType.DMA((2,)),
            pltpu.SemaphoreType.DMA((2,)),
        ],
    )
    return k(a1p, a2p, ftr)


def kernel(a1, a2, ft):
    a1r = a1.reshape(N, H)
    a1p = jnp.concatenate([a1r, a1r], axis=1)
    a2p = a2.reshape(N, A2W)
    ftr = ft.reshape(N, FTW)
    out = _sc_call(a1p, a2p, ftr)
    return out.reshape(N, H, DH)


# speedup vs baseline: 1.7608x; 1.7608x over previous
import functools
import jax
import jax.numpy as jnp
from jax import lax
from jax.experimental import pallas as pl
from jax.experimental.pallas import tpu as pltpu
from jax.experimental.pallas import tpu_sc as plsc

N = 10000
DEG = 32
H = 8
DH = 16
HDH = H * DH
FTW = DEG * HDH
A2W = DEG * H

NWORK = 32
C = 8
SPAN = 320
NC = SPAN // C


def _sc_body(a1_hbm, a2_hbm, ft_hbm, out_hbm,
             ftb, a2b, a1b, outb, rsc, insem, osem):
    cid = lax.axis_index("c")
    sid = lax.axis_index("s")
    wid = sid * 2 + cid
    start = 8 * ((wid * (N - SPAN)) // ((NWORK - 1) * 8))

    idx8 = lax.rem(lax.iota(jnp.int32, 16) + 8, 16)

    def issue_in(chunk, b):
        base = start + chunk * C
        pltpu.make_async_copy(
            ft_hbm.at[pl.ds(base, C), :], ftb.at[b], insem.at[b]).start()
        pltpu.make_async_copy(
            a2_hbm.at[pl.ds(base, C), :], a2b.at[b], insem.at[b]).start()
        pltpu.make_async_copy(
            a1_hbm.at[pl.ds(base, C), :], a1b.at[b], insem.at[b]).start()

    def wait_in(b):
        pltpu.make_async_copy(
            ft_hbm.at[pl.ds(0, C), :], ftb.at[b], insem.at[b]).wait()
        pltpu.make_async_copy(
            a2_hbm.at[pl.ds(0, C), :], a2b.at[b], insem.at[b]).wait()
        pltpu.make_async_copy(
            a1_hbm.at[pl.ds(0, C), :], a1b.at[b], insem.at[b]).wait()

    def compute_node(b, i):
        a1v = a1b[b, i, :]
        u = []
        for k in range(16):
            v = a2b[b, i, pl.ds(16 * k, 16)] + a1v
            u.append(jnp.maximum(v, 0.01 * v))
        m = u[0]
        for k in range(1, 16):
            m = jnp.maximum(m, u[k])
        rsc[:] = m
        m = jnp.maximum(m, plsc.load_gather(rsc, [idx8]))
        e = [jnp.exp(u[k] - m) for k in range(16)]
        s = e[0]
        for k in range(1, 16):
            s = s + e[k]
        rsc[:] = s
        s = s + plsc.load_gather(rsc, [idx8])
        r = 1.0 / s
        w = [e[k] * r for k in range(16)]
        for h in range(H):
            acc = w[0][h] * ftb[b, i, pl.ds(h * DH, 16)]
            for d in range(1, DEG):
                ws = w[d // 2][h + 8 * (d % 2)]
                acc = acc + ws * ftb[b, i, pl.ds(d * HDH + h * DH, 16)]
            outb[b, i, pl.ds(h * DH, 16)] = acc

    issue_in(0, 0)

    def outer(it0, carry):
        for b in range(2):
            chunk = it0 * 2 + b

            @pl.when(chunk + 1 < NC)
            def _():
                issue_in(chunk + 1, 1 - b)

            wait_in(b)

            @pl.when(chunk >= 2)
            def _():
                pltpu.make_async_copy(
                    outb.at[b], out_hbm.at[pl.ds(0, C), :], osem.at[b]).wait()

            def node_body(i, c):
                compute_node(b, i)
                return c
            lax.fori_loop(0, C, node_body, 0)

            base = start + chunk * C
            pltpu.make_async_copy(
                outb.at[b], out_hbm.at[pl.ds(base, C), :], osem.at[b]).start()
        return carry

    lax.fori_loop(0, NC // 2, outer, 0)
    for b in range(2):
        pltpu.make_async_copy(
            outb.at[b], out_hbm.at[pl.ds(0, C), :], osem.at[b]).wait()


@functools.partial(jax.jit, static_argnums=())
def _sc_call(a1p, a2p, ftr):
    mesh = plsc.VectorSubcoreMesh(core_axis_name="c", subcore_axis_name="s")
    k = pl.kernel(
        _sc_body,
        out_type=jax.ShapeDtypeStruct((N, HDH), jnp.float32),
        mesh=mesh,
        compiler_params=pltpu.CompilerParams(needs_layout_passes=False),
        scratch_types=[
            pltpu.VMEM((2, C, FTW), jnp.float32),
            pltpu.VMEM((2, C, A2W), jnp.float32),
            pltpu.VMEM((2, C, 16), jnp.float32),
            pltpu.VMEM((2, C, HDH), jnp.float32),
            pltpu.VMEM((16,), jnp.float32),
            pltpu.SemaphoreType.DMA((2,)),
            pltpu.SemaphoreType.DMA((2,)),
        ],
    )
    return k(a1p, a2p, ftr)


def kernel(a1, a2, ft):
    a1r = a1.reshape(N, H)
    a1p = jnp.concatenate([a1r, a1r], axis=1)
    a2p = a2.reshape(N, A2W)
    ftr = ft.reshape(N, FTW)
    out = _sc_call(a1p, a2p, ftr)
    return out.reshape(N, H, DH)
